# baseline (speedup 1.0000x reference)
import functools

import jax
import jax.numpy as jnp
from jax import lax
from jax.experimental import pallas as pl
from jax.experimental.pallas import tpu as pltpu
from jax.experimental.pallas import tpu_sc as plsc

_BOUNDARIES = tuple(float(v) for v in range(1, 100, 10))
_EPS = 1e-3
_B = 16384
_NU = 101
_NA = 11
_L = 16

_NC = 2
_NS = 16
_NW = _NC * _NS
_BPW = _B // _NW
_IDX_ROWS = _BPW // 128


def _table_body(tabT, age_tab, g1, b1, m1, v1, W1T, bias1,
                g2, b2, m2, v2, W2T, bias2, out_ref):
    s1 = g1[...] * lax.rsqrt(v1[...] + _EPS)
    t1 = b1[...] - m1[...] * s1
    w1t = W1T[...]
    w1u = w1t[:, :32] * s1[:, :32]
    w1a = w1t[:, 32:] * s1[:, 32:]
    dn_t = (((0,), (1,)), ((), ()))
    dn_r = (((1,), (1,)), ((), ()))
    tbu = lax.dot_general(t1[:, :32], w1t[:, :32], dn_r,
                          preferred_element_type=jnp.float32)
    tba = lax.dot_general(t1[:, 32:], w1t[:, 32:], dn_r,
                          preferred_element_type=jnp.float32)
    u = lax.dot_general(tabT[...], w1u, dn_t,
                        preferred_element_type=jnp.float32) + tbu + bias1[...]
    a = lax.dot_general(age_tab[...], w1a, dn_r,
                        preferred_element_type=jnp.float32) + tba
    s2 = g2[...] * lax.rsqrt(v2[...] + _EPS)
    t2 = b2[...] - m2[...] * s2
    for ab in range(_NA):
        h = jnp.maximum(u + a[ab:ab + 1, :], 0.0)
        g = h * s2 + t2
        y = lax.dot_general(g, W2T[...], dn_r,
                            preferred_element_type=jnp.float32) + bias2[...]
        out_ref[pl.ds(ab * _NU, _NU), :] = jnp.maximum(y, 0.0)


_table_call = pl.pallas_call(
    lambda *refs: _table_body(*refs),
    out_shape=jax.ShapeDtypeStruct((_NA * _NU, 16), jnp.float32),
)


def _sc_body(uid_hbm, age_hbm, tab_hbm, out_hbm, uid_v, age_v, idx_v, rows_v,
             sem_in, sem_g):
    wid = lax.axis_index("s") * _NC + lax.axis_index("c")
    base = wid * _BPW
    cp_u = pltpu.async_copy(uid_hbm.at[pl.ds(base, _BPW)], uid_v, sem_in)
    cp_a = pltpu.async_copy(age_hbm.at[pl.ds(base, _BPW)], age_v, sem_in)
    cp_u.wait()
    cp_a.wait()
    gathers = []
    for j in range(_IDX_ROWS):
        for i in range(128 // _L):
            off = j * 128 + i * _L
            a = age_v[pl.ds(off, _L)]
            u = uid_v[pl.ds(off, _L)]
            b = jnp.zeros((_L,), jnp.int32)
            for c in _BOUNDARIES:
                b = b + jnp.where(a >= c, 1, 0)
            idx_v[j, pl.ds(i * _L, _L)] = b * _NU + u
        gathers.append(pltpu.async_copy(tab_hbm.at[idx_v.at[j]],
                                        rows_v.at[pl.ds(j * 128, 128)], sem_g))
    for cp in gathers:
        cp.wait()
    outs = [pltpu.async_copy(rows_v.at[:, pl.ds(j, 1)],
                             out_hbm.at[j, pl.ds(base, _BPW), pl.ds(0, 1)],
                             sem_in)
            for j in range(16)]
    for cp in outs:
        cp.wait()


@functools.cache
def _make_sc_call():
    return functools.partial(
        pl.kernel,
        mesh=plsc.VectorSubcoreMesh(core_axis_name="c", subcore_axis_name="s"),
        compiler_params=pltpu.CompilerParams(use_tc_tiling_on_sc=False),
        out_type=jax.ShapeDtypeStruct((16, _B, 8), jnp.float32),
        scratch_types=[
            pltpu.VMEM((_BPW,), jnp.int32),
            pltpu.VMEM((_BPW,), jnp.float32),
            pltpu.VMEM((_IDX_ROWS, 128), jnp.int32),
            pltpu.VMEM((_BPW, 16), jnp.float32),
            pltpu.SemaphoreType.DMA,
            pltpu.SemaphoreType.DMA,
        ],
    )(_sc_body)


def kernel(user_id, item_id, price, user_age, user_id_table, age_table,
           bn1_gamma, bn1_beta, bn1_mean, bn1_var, W1, b1,
           bn2_gamma, bn2_beta, bn2_mean, bn2_var, W2, b2):
    del item_id, price
    row = lambda x: x.reshape(1, -1)
    tab = _table_call(user_id_table.T, age_table,
                      row(bn1_gamma), row(bn1_beta), row(bn1_mean), row(bn1_var),
                      W1.T, row(b1),
                      row(bn2_gamma), row(bn2_beta), row(bn2_mean), row(bn2_var),
                      W2.T, row(b2))
    out3 = _make_sc_call()(user_id.astype(jnp.int32), user_age, tab)
    return out3[:, :, 0].T


# speedup vs baseline: 17.1426x; 17.1426x over previous
import functools

import jax
import jax.numpy as jnp
from jax import lax
from jax.experimental import pallas as pl
from jax.experimental.pallas import tpu as pltpu
from jax.experimental.pallas import tpu_sc as plsc

_BOUNDARIES = tuple(float(v) for v in range(1, 100, 10))
_EPS = 1e-3
_B = 16384
_NU = 101
_NA = 11
_L = 16

_NC = 2
_NS = 16
_NW = _NC * _NS
_BPW = _B // _NW
_IDX_ROWS = _BPW // 128


def _table_body(tabT, age_tab, g1, b1, m1, v1, W1T, bias1,
                g2, b2, m2, v2, W2T, bias2, out_ref):
    uid_tab = jnp.transpose(tabT[...])
    W1 = jnp.transpose(W1T[...])
    W2 = jnp.transpose(W2T[...])
    s1 = g1[...] * lax.rsqrt(v1[...] + _EPS)
    t1 = b1[...] - m1[...] * s1
    uq = uid_tab * s1[:, :32] + t1[:, :32]
    aq = age_tab[...] * s1[:, 32:] + t1[:, 32:]
    u = jnp.dot(uq, W1[:32, :], preferred_element_type=jnp.float32) + bias1[...]
    a = jnp.dot(aq, W1[32:, :], preferred_element_type=jnp.float32)
    s2 = g2[...] * lax.rsqrt(v2[...] + _EPS)
    t2 = b2[...] - m2[...] * s2
    for ab in range(_NA):
        h = jnp.maximum(u + a[ab:ab + 1, :], 0.0)
        g = h * s2 + t2
        y = jnp.dot(g, W2, preferred_element_type=jnp.float32) + bias2[...]
        out_ref[pl.ds(ab * _NU, _NU), :] = jnp.maximum(y, 0.0)


_table_call = pl.pallas_call(
    lambda *refs: _table_body(*refs),
    out_shape=jax.ShapeDtypeStruct((_NA * _NU, 16), jnp.float32),
)


def _sc_body(uid_hbm, age_hbm, tab_hbm, out_hbm, uid_v, age_v, idx_v, rows_v,
             sem_in, sem_g):
    wid = lax.axis_index("s") * _NC + lax.axis_index("c")
    base = wid * _BPW
    cp_u = pltpu.async_copy(uid_hbm.at[pl.ds(base, _BPW)], uid_v, sem_in)
    cp_a = pltpu.async_copy(age_hbm.at[pl.ds(base, _BPW)], age_v, sem_in)
    cp_u.wait()
    cp_a.wait()
    gathers = []
    for j in range(_IDX_ROWS):
        for i in range(128 // _L):
            off = j * 128 + i * _L
            a = age_v[pl.ds(off, _L)]
            u = uid_v[pl.ds(off, _L)]
            b = jnp.zeros((_L,), jnp.int32)
            for c in _BOUNDARIES:
                b = b + jnp.where(a >= c, 1, 0)
            idx_v[j, pl.ds(i * _L, _L)] = b * _NU + u
        gathers.append(pltpu.async_copy(tab_hbm.at[idx_v.at[j]],
                                        rows_v.at[pl.ds(j * 128, 128)], sem_g))
    for cp in gathers:
        cp.wait()
    pltpu.sync_copy(rows_v, out_hbm.at[pl.ds(base, _BPW)])


@functools.cache
def _make_sc_call():
    return functools.partial(
        pl.kernel,
        mesh=plsc.VectorSubcoreMesh(core_axis_name="c", subcore_axis_name="s"),
        compiler_params=pltpu.CompilerParams(use_tc_tiling_on_sc=False),
        out_type=jax.ShapeDtypeStruct((_B, 16), jnp.float32),
        scratch_types=[
            pltpu.VMEM((_BPW,), jnp.int32),
            pltpu.VMEM((_BPW,), jnp.float32),
            pltpu.VMEM((_IDX_ROWS, 128), jnp.int32),
            pltpu.VMEM((_BPW, 16), jnp.float32),
            pltpu.SemaphoreType.DMA,
            pltpu.SemaphoreType.DMA,
        ],
    )(_sc_body)


def kernel(user_id, item_id, price, user_age, user_id_table, age_table,
           bn1_gamma, bn1_beta, bn1_mean, bn1_var, W1, b1,
           bn2_gamma, bn2_beta, bn2_mean, bn2_var, W2, b2):
    del item_id, price
    row = lambda x: x.reshape(1, -1)
    tab = _table_call(user_id_table.T, age_table,
                      row(bn1_gamma), row(bn1_beta), row(bn1_mean), row(bn1_var),
                      W1.T, row(b1),
                      row(bn2_gamma), row(bn2_beta), row(bn2_mean), row(bn2_var),
                      W2.T, row(b2))
    return _make_sc_call()(user_id.astype(jnp.int32), user_age, tab)


# speedup vs baseline: 17.7316x; 1.0344x over previous
import functools

import jax
import jax.numpy as jnp
from jax import lax
from jax.experimental import pallas as pl
from jax.experimental.pallas import tpu as pltpu
from jax.experimental.pallas import tpu_sc as plsc

_BOUNDARIES = tuple(float(v) for v in range(1, 100, 10))
_EPS = 1e-3
_B = 16384
_NU = 101
_NA = 11
_L = 16

_NC = 2
_NS = 16
_NW = _NC * _NS
_BPW = _B // _NW
_IDX_ROWS = _BPW // 128


def _table_body(tabT, age_tab, g1, b1, m1, v1, W1T, bias1,
                g2, b2, m2, v2, W2T, bias2, out_ref):
    uid_tab = jnp.transpose(tabT[...])
    W1 = jnp.transpose(W1T[...])
    W2 = jnp.transpose(W2T[...])
    s1 = g1[...] * lax.rsqrt(v1[...] + _EPS)
    t1 = b1[...] - m1[...] * s1
    uq = uid_tab * s1[:, :32] + t1[:, :32]
    aq = age_tab[...] * s1[:, 32:] + t1[:, 32:]
    u = jnp.dot(uq, W1[:32, :], preferred_element_type=jnp.float32) + bias1[...]
    a = jnp.dot(aq, W1[32:, :], preferred_element_type=jnp.float32)
    s2 = g2[...] * lax.rsqrt(v2[...] + _EPS)
    t2 = b2[...] - m2[...] * s2
    for ab in range(_NA):
        h = jnp.maximum(u + a[ab:ab + 1, :], 0.0)
        g = h * s2 + t2
        y = jnp.dot(g, W2, preferred_element_type=jnp.float32) + bias2[...]
        out_ref[pl.ds((ab >> 3) * 104, _NU),
                pl.ds((ab & 7) * 16, 16)] = jnp.maximum(y, 0.0)


_table_call = pl.pallas_call(
    lambda *refs: _table_body(*refs),
    out_shape=jax.ShapeDtypeStruct((208, 128), jnp.float32),
)


def _sc_body(uid_hbm, age_hbm, tab_hbm, out_hbm, uid_v, age_v, idx_v, rows_v,
             sem_in, sem_g):
    wid = lax.axis_index("s") * _NC + lax.axis_index("c")
    base = wid * _BPW
    cp_u = pltpu.async_copy(uid_hbm.at[pl.ds(base, _BPW)], uid_v, sem_in)
    cp_a = pltpu.async_copy(age_hbm.at[pl.ds(base, _BPW)], age_v, sem_in)
    cp_u.wait()
    cp_a.wait()
    gathers = []
    for j in range(_IDX_ROWS):
        for i in range(128 // _L):
            off = j * 128 + i * _L
            a = age_v[pl.ds(off, _L)]
            u = uid_v[pl.ds(off, _L)]
            b = jnp.zeros((_L,), jnp.int32)
            for c in _BOUNDARIES:
                b = b + jnp.where(a >= c, 1, 0)
            idx_v[j, pl.ds(i * _L, _L)] = u * 8 + b + jnp.where(b >= 8, 824, 0)
        gathers.append(pltpu.async_copy(tab_hbm.at[idx_v.at[j]],
                                        rows_v.at[pl.ds(j * 128, 128)], sem_g))
    for cp in gathers:
        cp.wait()
    pltpu.sync_copy(rows_v, out_hbm.at[pl.ds(base, _BPW)])


@functools.cache
def _make_sc_call():
    return functools.partial(
        pl.kernel,
        mesh=plsc.VectorSubcoreMesh(core_axis_name="c", subcore_axis_name="s"),
        compiler_params=pltpu.CompilerParams(use_tc_tiling_on_sc=False),
        out_type=jax.ShapeDtypeStruct((_B, 16), jnp.float32),
        scratch_types=[
            pltpu.VMEM((_BPW,), jnp.int32),
            pltpu.VMEM((_BPW,), jnp.float32),
            pltpu.VMEM((_IDX_ROWS, 128), jnp.int32),
            pltpu.VMEM((_BPW, 16), jnp.float32),
            pltpu.SemaphoreType.DMA,
            pltpu.SemaphoreType.DMA,
        ],
    )(_sc_body)


def kernel(user_id, item_id, price, user_age, user_id_table, age_table,
           bn1_gamma, bn1_beta, bn1_mean, bn1_var, W1, b1,
           bn2_gamma, bn2_beta, bn2_mean, bn2_var, W2, b2):
    del item_id, price
    row = lambda x: x.reshape(1, -1)
    tab = _table_call(user_id_table.T, age_table,
                      row(bn1_gamma), row(bn1_beta), row(bn1_mean), row(bn1_var),
                      W1.T, row(b1),
                      row(bn2_gamma), row(bn2_beta), row(bn2_mean), row(bn2_var),
                      W2.T, row(b2))
    return _make_sc_call()(user_id.astype(jnp.int32), user_age,
                           tab.reshape(1664, 16))


# speedup vs baseline: 17.7760x; 1.0025x over previous
import functools

import jax
import jax.numpy as jnp
from jax import lax
from jax.experimental import pallas as pl
from jax.experimental.pallas import tpu as pltpu
from jax.experimental.pallas import tpu_sc as plsc

_BOUNDARIES = tuple(float(v) for v in range(1, 100, 10))
_EPS = 1e-3
_B = 16384
_NU = 101
_NA = 11
_L = 16

_NC = 2
_NS = 16
_NW = _NC * _NS
_BPW = _B // _NW
_IDX_ROWS = _BPW // 128


def _table_body(tabT, age_tab, g1, b1, m1, v1, W1T, bias1,
                g2, b2, m2, v2, W2T, bias2, out_ref):
    uid_tab = jnp.transpose(tabT[...])
    W1 = jnp.transpose(W1T[...])
    W2 = jnp.transpose(W2T[...])
    s1 = g1[...] * lax.rsqrt(v1[...] + _EPS)
    t1 = b1[...] - m1[...] * s1
    uq = uid_tab * s1[:, :32] + t1[:, :32]
    aq = age_tab[...] * s1[:, 32:] + t1[:, 32:]
    u = jnp.dot(uq, W1[:32, :], preferred_element_type=jnp.float32) + bias1[...]
    a = jnp.dot(aq, W1[32:, :], preferred_element_type=jnp.float32)
    s2 = g2[...] * lax.rsqrt(v2[...] + _EPS)
    t2 = b2[...] - m2[...] * s2
    for ab in range(_NA):
        h = jnp.maximum(u + a[ab:ab + 1, :], 0.0)
        g = h * s2 + t2
        y = jnp.dot(g, W2, preferred_element_type=jnp.float32) + bias2[...]
        out_ref[pl.ds((ab >> 3) * 104, _NU),
                pl.ds((ab & 7) * 16, 16)] = jnp.maximum(y, 0.0)


_table_call = pl.pallas_call(
    lambda *refs: _table_body(*refs),
    out_shape=jax.ShapeDtypeStruct((208, 128), jnp.float32),
)


def _sc_body(uid_hbm, age_hbm, tab_hbm, out_hbm, uid_v, age_v, idx_v, rows_v,
             sem_in, sem_g):
    wid = lax.axis_index("s") * _NC + lax.axis_index("c")
    base = wid * _BPW
    cp_u = pltpu.async_copy(uid_hbm.at[pl.ds(base, _BPW)], uid_v, sem_in)
    cp_a = pltpu.async_copy(age_hbm.at[pl.ds(base, _BPW)], age_v, sem_in)
    cp_u.wait()
    cp_a.wait()
    gathers = []
    for j in range(_IDX_ROWS):
        for i in range(128 // _L):
            off = j * 128 + i * _L
            a = age_v[pl.ds(off, _L)]
            u = uid_v[pl.ds(off, _L)]
            b = jnp.zeros((_L,), jnp.int32)
            for c in _BOUNDARIES:
                b = b + jnp.where(a >= c, 1, 0)
            idx_v[j, pl.ds(i * _L, _L)] = u * 8 + b + jnp.where(b >= 8, 824, 0)
        gathers.append(pltpu.async_copy(tab_hbm.at[idx_v.at[j]],
                                        rows_v.at[pl.ds(j * 128, 128)], sem_g))
    outs = []
    for g, cp in enumerate(gathers):
        cp.wait()
        outs.append(pltpu.async_copy(rows_v.at[pl.ds(g * 128, 128)],
                                     out_hbm.at[pl.ds(base + g * 128, 128)],
                                     sem_in))
    for cp in outs:
        cp.wait()


@functools.cache
def _make_sc_call():
    return functools.partial(
        pl.kernel,
        mesh=plsc.VectorSubcoreMesh(core_axis_name="c", subcore_axis_name="s"),
        compiler_params=pltpu.CompilerParams(use_tc_tiling_on_sc=False),
        out_type=jax.ShapeDtypeStruct((_B, 16), jnp.float32),
        scratch_types=[
            pltpu.VMEM((_BPW,), jnp.int32),
            pltpu.VMEM((_BPW,), jnp.float32),
            pltpu.VMEM((_IDX_ROWS, 128), jnp.int32),
            pltpu.VMEM((_BPW, 16), jnp.float32),
            pltpu.SemaphoreType.DMA,
            pltpu.SemaphoreType.DMA,
        ],
    )(_sc_body)


def kernel(user_id, item_id, price, user_age, user_id_table, age_table,
           bn1_gamma, bn1_beta, bn1_mean, bn1_var, W1, b1,
           bn2_gamma, bn2_beta, bn2_mean, bn2_var, W2, b2):
    del item_id, price
    row = lambda x: x.reshape(1, -1)
    tab = _table_call(user_id_table.T, age_table,
                      row(bn1_gamma), row(bn1_beta), row(bn1_mean), row(bn1_var),
                      W1.T, row(b1),
                      row(bn2_gamma), row(bn2_beta), row(bn2_mean), row(bn2_var),
                      W2.T, row(b2))
    return _make_sc_call()(user_id.astype(jnp.int32), user_age,
                           tab.reshape(1664, 16))
